# baseline (speedup 1.0000x reference)
import jax
import jax.numpy as jnp
from jax import lax
from jax.experimental import pallas as pl
from jax.experimental.pallas import tpu as pltpu
from jax.experimental.pallas import tpu_sc as plsc

NC = 2
NS = 16

CHUNK = 128
N_PAD = 10112
ROWS_PER_TILE = N_PAD // NS

Q0 = 104
Q1 = 56


def _prenorm_body(feat_ref, onorm_ref, o_ref):
    o_ref[...] = feat_ref[...] / onorm_ref[...]


def _postnorm_matmul_body(parts_ref, inorm_ref, w_ref, b_ref, o_ref):
    h2 = (parts_ref[0] + parts_ref[1]) / inorm_ref[...]
    o_ref[...] = jax.lax.dot_general(
        h2, w_ref[...], (((1,), (1,)), ((), ())),
        preferred_element_type=jnp.float32) + b_ref[...]


def _edge_agg_body(src_hbm, dst_hbm, hp_hbm, zero_hbm, part_hbm,
                   src_v, dst_v, rows_v, agg_sh, sem):
    c = lax.axis_index("c")
    s = lax.axis_index("s")

    t0 = s * ROWS_PER_TILE
    pltpu.sync_copy(zero_hbm.at[pl.ds(t0, ROWS_PER_TILE)],
                    agg_sh.at[pl.ds(t0, ROWS_PER_TILE)])

    def process(start, q):
        pltpu.sync_copy(src_hbm.at[pl.ds(start, q)], src_v.at[pl.ds(0, q)])
        pltpu.sync_copy(dst_hbm.at[pl.ds(start, q)], dst_v.at[pl.ds(0, q)])

        def body(m, carry):
            pltpu.async_copy(hp_hbm.at[src_v.at[m]], rows_v, sem).wait()
            pltpu.sync_copy(rows_v, agg_sh.at[dst_v.at[m]], add=True)
            return carry

        lax.fori_loop(0, q, body, 0)

    @pl.when(c == 1)
    def _():
        process(s * Q1, Q1)

    @pl.when(c == 0)
    def _():
        process(NS * Q1 + s * Q0, Q0)

    plsc.subcore_barrier()
    pltpu.sync_copy(agg_sh.at[pl.ds(t0, ROWS_PER_TILE)],
                    part_hbm.at[c, pl.ds(t0, ROWS_PER_TILE)])


def _edge_aggregate(src2, dst2, hp, zero):
    mesh = plsc.VectorSubcoreMesh(core_axis_name="c", subcore_axis_name="s")
    return pl.kernel(
        _edge_agg_body,
        out_type=jax.ShapeDtypeStruct((NC, N_PAD, 128), jnp.float32),
        mesh=mesh,
        scratch_types=[
            pltpu.VMEM((Q0, CHUNK), jnp.int32),
            pltpu.VMEM((Q0, CHUNK), jnp.int32),
            pltpu.VMEM((CHUNK, 128), jnp.float32),
            pltpu.VMEM_SHARED((N_PAD, 128), jnp.float32),
            pltpu.SemaphoreType.DMA,
        ],
    )(src2, dst2, hp, zero)


@jax.jit
def kernel(feat, edge_index, in_norm, out_norm, W, b):
    n, d_in = feat.shape
    e = edge_index.shape[1]

    pad_n = N_PAD - n
    feat_p = jnp.pad(feat, ((0, pad_n), (0, 0)))
    onorm_p = jnp.pad(out_norm, (0, pad_n), constant_values=1.0)[:, None]
    inorm_p = jnp.pad(in_norm, (0, pad_n), constant_values=1.0)[:, None]

    e_pad = NS * (Q0 + Q1) * CHUNK
    src = edge_index[0]
    dst = edge_index[1]
    npad_e = e_pad - e
    src_p = jnp.concatenate([src, jnp.zeros((npad_e,), jnp.int32)])
    dst_p = jnp.concatenate(
        [dst, n + (jnp.arange(npad_e, dtype=jnp.int32) % (N_PAD - n))])
    src2 = src_p.reshape(-1, CHUNK)
    dst2 = dst_p.reshape(-1, CHUNK)

    zero = jnp.zeros((N_PAD, 128), jnp.float32)

    hp = pl.pallas_call(
        _prenorm_body,
        out_shape=jax.ShapeDtypeStruct((N_PAD, 128), jnp.float32),
    )(feat_p, onorm_p)

    parts = _edge_aggregate(src2, dst2, hp, zero)

    out = pl.pallas_call(
        _postnorm_matmul_body,
        out_shape=jax.ShapeDtypeStruct((N_PAD, 128), jnp.float32),
    )(parts, inorm_p, W, b[None, :])

    return out[:n]


# speedup vs baseline: 4.0355x; 4.0355x over previous
import jax
import jax.numpy as jnp
from jax import lax
from jax.experimental import pallas as pl
from jax.experimental.pallas import tpu as pltpu
from jax.experimental.pallas import tpu_sc as plsc

NC = 2
NS = 16
NW = NC * NS

CHUNK = 128
N_PAD = 10112
ROWS_PER_TILE = N_PAD // NS

NBUF = 2
PHASE = 40
Q_BIG = 80
Q_SMALL = 72
N_BIG = 25


def _prenorm_body(feat_ref, onorm_ref, o_ref):
    o_ref[...] = feat_ref[...] / onorm_ref[...]


def _postnorm_matmul_body(parts_ref, inorm_ref, w_ref, b_ref, o_ref):
    h2 = (parts_ref[0] + parts_ref[1]) / inorm_ref[...]
    o_ref[...] = jax.lax.dot_general(
        h2, w_ref[...], (((1,), (1,)), ((), ())),
        preferred_element_type=jnp.float32) + b_ref[...]


def _edge_agg_body(src_hbm, dst_hbm, hp_hbm, zero_hbm, part_hbm,
                   src_v, dst_v, rows_v, agg_sh, *sems):
    c = lax.axis_index("c")
    s = lax.axis_index("s")
    w = c * NS + s

    t0 = s * ROWS_PER_TILE
    pltpu.sync_copy(zero_hbm.at[pl.ds(t0, ROWS_PER_TILE)],
                    agg_sh.at[pl.ds(t0, ROWS_PER_TILE)])

    def pipeline(phase_start, q):
        pltpu.sync_copy(src_hbm.at[pl.ds(phase_start, PHASE)], src_v)
        pltpu.sync_copy(dst_hbm.at[pl.ds(phase_start, PHASE)], dst_v)

        for b in range(NBUF):
            pltpu.async_copy(hp_hbm.at[src_v.at[b]], rows_v.at[b], sems[b])

        def body(i, carry):
            j = i * NBUF
            for b in range(NBUF):
                m = j + b
                pltpu.make_async_copy(hp_hbm.at[src_v.at[m]], rows_v.at[b],
                                      sems[b]).wait()
                pltpu.sync_copy(rows_v.at[b], agg_sh.at[dst_v.at[m]],
                                add=True)
                nm = m + NBUF

                @pl.when(nm < q)
                def _():
                    pltpu.async_copy(hp_hbm.at[src_v.at[nm]], rows_v.at[b],
                                     sems[b])
            return carry

        lax.fori_loop(0, q // NBUF, body, 0)

    def process(start, q_static):
        for p in range(0, q_static, PHASE):
            pipeline(start + p, min(PHASE, q_static - p))

    @pl.when(w < N_BIG)
    def _():
        process(w * Q_BIG, Q_BIG)

    @pl.when(w >= N_BIG)
    def _():
        process(N_BIG * Q_BIG + (w - N_BIG) * Q_SMALL, Q_SMALL)

    plsc.subcore_barrier()
    pltpu.sync_copy(agg_sh.at[pl.ds(t0, ROWS_PER_TILE)],
                    part_hbm.at[c, pl.ds(t0, ROWS_PER_TILE)])


def _edge_aggregate(src2, dst2, hp, zero):
    mesh = plsc.VectorSubcoreMesh(core_axis_name="c", subcore_axis_name="s")
    return pl.kernel(
        _edge_agg_body,
        out_type=jax.ShapeDtypeStruct((NC, N_PAD, 128), jnp.float32),
        mesh=mesh,
        scratch_types=[
            pltpu.VMEM((PHASE, CHUNK), jnp.int32),
            pltpu.VMEM((PHASE, CHUNK), jnp.int32),
            pltpu.VMEM((NBUF, CHUNK, 128), jnp.float32),
            pltpu.VMEM_SHARED((N_PAD, 128), jnp.float32),
        ] + [pltpu.SemaphoreType.DMA] * NBUF,
    )(src2, dst2, hp, zero)


@jax.jit
def kernel(feat, edge_index, in_norm, out_norm, W, b):
    n, d_in = feat.shape
    e = edge_index.shape[1]

    pad_n = N_PAD - n
    feat_p = jnp.pad(feat, ((0, pad_n), (0, 0)))
    onorm_p = jnp.pad(out_norm, (0, pad_n), constant_values=1.0)[:, None]
    inorm_p = jnp.pad(in_norm, (0, pad_n), constant_values=1.0)[:, None]

    e_proc = (N_BIG * Q_BIG + (NW - N_BIG) * Q_SMALL) * CHUNK
    npad_e = e_proc - e
    src = edge_index[0]
    dst = edge_index[1]
    idx_pad = jnp.arange(npad_e + 8 * CHUNK, dtype=jnp.int32)
    src_p = jnp.concatenate([src, idx_pad % n])
    dst_p = jnp.concatenate([dst, n + idx_pad % (N_PAD - n)])
    src2 = src_p.reshape(-1, CHUNK)
    dst2 = dst_p.reshape(-1, CHUNK)

    zero = jnp.zeros((N_PAD, 128), jnp.float32)

    hp = pl.pallas_call(
        _prenorm_body,
        out_shape=jax.ShapeDtypeStruct((N_PAD, 128), jnp.float32),
    )(feat_p, onorm_p)

    parts = _edge_aggregate(src2, dst2, hp, zero)

    out = pl.pallas_call(
        _postnorm_matmul_body,
        out_shape=jax.ShapeDtypeStruct((N_PAD, 128), jnp.float32),
    )(parts, inorm_p, W, b[None, :])

    return out[:n]


# speedup vs baseline: 4.3630x; 1.0812x over previous
import jax
import jax.numpy as jnp
from jax import lax
from jax.experimental import pallas as pl
from jax.experimental.pallas import tpu as pltpu
from jax.experimental.pallas import tpu_sc as plsc

NC = 2
NS = 16
NW = NC * NS

CHUNK = 128
N_PAD = 10112
ROWS_PER_TILE = N_PAD // NS

NBUF = 2
PHASE = 40
Q_MAIN = 80
MAIN_CHUNKS = 31 * Q_MAIN + 16
TAIL_PROC = 8


def _prenorm_body(feat_ref, onorm_ref, o_ref):
    o_ref[...] = feat_ref[...] / onorm_ref[...]


def _postnorm_matmul_body(parts_ref, inorm_ref, w_ref, b_ref, o_ref):
    n = o_ref.shape[0]
    h2 = (parts_ref[0, :n, :] + parts_ref[1, :n, :]) / inorm_ref[...]
    o_ref[...] = jax.lax.dot_general(
        h2, w_ref[...], (((1,), (1,)), ((), ())),
        preferred_element_type=jnp.float32) + b_ref[...]


def _edge_agg_body(src_hbm, dst_hbm, tsrc_hbm, tdst_hbm, hp_hbm, zero_hbm,
                   part_hbm, src_v, dst_v, rows_v, agg_sh, *sems):
    c = lax.axis_index("c")
    s = lax.axis_index("s")
    w = c * NS + s

    t0 = s * ROWS_PER_TILE
    pltpu.sync_copy(zero_hbm, agg_sh.at[pl.ds(t0, ROWS_PER_TILE)])

    def pipeline(sref, dref, start, q):
        pltpu.sync_copy(sref.at[pl.ds(start, q)], src_v.at[pl.ds(0, q)])
        pltpu.sync_copy(dref.at[pl.ds(start, q)], dst_v.at[pl.ds(0, q)])

        for b in range(NBUF):
            pltpu.async_copy(hp_hbm.at[src_v.at[b]], rows_v.at[b], sems[b])

        def body(i, carry):
            j = i * NBUF
            for b in range(NBUF):
                m = j + b
                pltpu.make_async_copy(hp_hbm.at[src_v.at[m]], rows_v.at[b],
                                      sems[b]).wait()
                pltpu.sync_copy(rows_v.at[b], agg_sh.at[dst_v.at[m]],
                                add=True)
                nm = m + NBUF

                @pl.when(nm < q)
                def _():
                    pltpu.async_copy(hp_hbm.at[src_v.at[nm]], rows_v.at[b],
                                     sems[b])
            return carry

        lax.fori_loop(0, q // NBUF, body, 0)

    @pl.when(w < NW - 1)
    def _():
        for p in range(0, Q_MAIN, PHASE):
            pipeline(src_hbm, dst_hbm, w * Q_MAIN + p,
                     min(PHASE, Q_MAIN - p))

    @pl.when(w == NW - 1)
    def _():
        pipeline(src_hbm, dst_hbm, (NW - 1) * Q_MAIN,
                 MAIN_CHUNKS - (NW - 1) * Q_MAIN)
        pipeline(tsrc_hbm, tdst_hbm, 0, TAIL_PROC)

    plsc.subcore_barrier()
    pltpu.sync_copy(agg_sh.at[pl.ds(t0, ROWS_PER_TILE)],
                    part_hbm.at[c, pl.ds(t0, ROWS_PER_TILE)])


def _edge_aggregate(src2, dst2, tsrc, tdst, hp, zero):
    mesh = plsc.VectorSubcoreMesh(core_axis_name="c", subcore_axis_name="s")
    return pl.kernel(
        _edge_agg_body,
        out_type=jax.ShapeDtypeStruct((NC, N_PAD, 128), jnp.float32),
        mesh=mesh,
        scratch_types=[
            pltpu.VMEM((PHASE, CHUNK), jnp.int32),
            pltpu.VMEM((PHASE, CHUNK), jnp.int32),
            pltpu.VMEM((NBUF, CHUNK, 128), jnp.float32),
            pltpu.VMEM_SHARED((N_PAD, 128), jnp.float32),
        ] + [pltpu.SemaphoreType.DMA] * NBUF,
    )(src2, dst2, tsrc, tdst, hp, zero)


@jax.jit
def kernel(feat, edge_index, in_norm, out_norm, W, b):
    n, d_in = feat.shape
    e = edge_index.shape[1]

    e_main = MAIN_CHUNKS * CHUNK
    src2 = edge_index[0, :e_main].reshape(-1, CHUNK)
    dst2 = edge_index[1, :e_main].reshape(-1, CHUNK)
    n_tail = TAIL_PROC * CHUNK
    idx_pad = jnp.arange(n_tail - (e - e_main), dtype=jnp.int32)
    tsrc = jnp.concatenate([edge_index[0, e_main:], idx_pad % n])
    tdst = jnp.concatenate(
        [edge_index[1, e_main:], n + idx_pad % (N_PAD - n)])
    tsrc = tsrc.reshape(-1, CHUNK)
    tdst = tdst.reshape(-1, CHUNK)

    zero = jnp.zeros((ROWS_PER_TILE, 128), jnp.float32)

    hp = pl.pallas_call(
        _prenorm_body,
        out_shape=jax.ShapeDtypeStruct((n, 128), jnp.float32),
    )(feat, out_norm[:, None])

    parts = _edge_aggregate(src2, dst2, tsrc, tdst, hp, zero)

    out = pl.pallas_call(
        _postnorm_matmul_body,
        out_shape=jax.ShapeDtypeStruct((n, 128), jnp.float32),
    )(parts, in_norm[:, None], W, b[None, :])

    return out
